# Initial kernel scaffold; baseline (speedup 1.0000x reference)
import jax
import jax.numpy as jnp
from jax.experimental import pallas as pl


def kernel(x, rep0, rep1, rep2):
    raise NotImplementedError("write your pallas kernel here")


# speedup vs baseline: 7.3071x; 7.3071x over previous
import functools

import jax
import jax.numpy as jnp
from jax import lax
from jax.experimental import pallas as pl
from jax.experimental.pallas import tpu as pltpu
from jax.experimental.pallas import tpu_sc as plsc

G = 100000
B = 16384
D0, D1, D2 = 16, 64, 256
OUT_D = D0 + D1 + D2

_info = plsc.get_sparse_core_info()
NC, NS = _info.num_cores, _info.num_subcores
NW = NC * NS
BPW = B // NW
CH = 128
NCH = BPW // CH

_mesh = plsc.VectorSubcoreMesh(core_axis_name="c", subcore_axis_name="s")


@functools.partial(
    pl.kernel,
    mesh=_mesh,
    out_type=jax.ShapeDtypeStruct((B, OUT_D), jnp.float32),
    compiler_params=pltpu.CompilerParams(use_tc_tiling_on_sc=False),
    scratch_types=[
        pltpu.VMEM((NCH, CH), jnp.int32),
        pltpu.VMEM((BPW, D0), jnp.float32),
        pltpu.VMEM((BPW, D1), jnp.float32),
        pltpu.VMEM((BPW // 2, D2), jnp.float32),
        pltpu.SemaphoreType.DMA,
        pltpu.SemaphoreType.DMA,
        pltpu.SemaphoreType.DMA,
    ],
)
def _sc_gather(x_hbm, rep0_hbm, rep1_hbm, rep2_hbm, out_hbm,
               idx_v, rows0_v, rows1_v, rows2_v, sem0, sem1, sem2):
    wid = lax.axis_index("s") * NC + lax.axis_index("c")
    base = wid * BPW

    pltpu.sync_copy(x_hbm.at[pl.ds(wid * NCH, NCH)], idx_v)

    h2 = [
        pltpu.async_copy(rep2_hbm.at[idx_v.at[j]],
                         rows2_v.at[pl.ds(j * CH, CH)], sem2)
        for j in range(NCH // 2)
    ]
    h0 = [
        pltpu.async_copy(rep0_hbm.at[idx_v.at[j]],
                         rows0_v.at[pl.ds(j * CH, CH)], sem0)
        for j in range(NCH)
    ]
    h1 = [
        pltpu.async_copy(rep1_hbm.at[idx_v.at[j]],
                         rows1_v.at[pl.ds(j * CH, CH)], sem1)
        for j in range(NCH)
    ]

    for h in h2:
        h.wait()
    pltpu.sync_copy(rows2_v,
                    out_hbm.at[pl.ds(base, BPW // 2), pl.ds(D0 + D1, D2)])

    h2b = [
        pltpu.async_copy(rep2_hbm.at[idx_v.at[j]],
                         rows2_v.at[pl.ds((j - NCH // 2) * CH, CH)], sem2)
        for j in range(NCH // 2, NCH)
    ]

    for h in h0:
        h.wait()
    pltpu.sync_copy(rows0_v, out_hbm.at[pl.ds(base, BPW), pl.ds(0, D0)])
    for h in h1:
        h.wait()
    pltpu.sync_copy(rows1_v, out_hbm.at[pl.ds(base, BPW), pl.ds(D0, D1)])

    for h in h2b:
        h.wait()
    pltpu.sync_copy(rows2_v,
                    out_hbm.at[pl.ds(base + BPW // 2, BPW // 2),
                               pl.ds(D0 + D1, D2)])


def kernel(x, rep0, rep1, rep2):
    x2 = x.astype(jnp.int32).reshape(B // CH, CH)
    return _sc_gather(x2, rep0.reshape(G, D0), rep1.reshape(G, D1),
                      rep2.reshape(G, D2))
